# baseline (speedup 1.0000x reference)
import jax
import jax.numpy as jnp
from jax import lax
from jax.experimental import pallas as pl
from jax.experimental.pallas import tpu as pltpu
from jax.experimental.pallas import tpu_sc as plsc

N_NODES = 10000
DIM = 128
N_EDGES = 320000

NC = 2
NS = 16
NW = NC * NS

EDGES_PER_W = N_EDGES // NW
CHUNK = 128
N_CHUNKS = (EDGES_PER_W + CHUNK - 1) // CHUNK
BLK = 16
DEPTH = 3


def _dot_chunk(zr_ref, zc_ref, pbuf, out_ref):
    nblk = CHUNK // BLK
    nseg = DIM // 16
    col_ids = [lax.iota(jnp.int32, 16) * BLK + l for l in range(BLK)]

    def block(b, carry):
        base_e = b * BLK
        for e in range(BLK):
            row = base_e + e
            prods = [
                zr_ref[row, pl.ds(l * 16, 16)] * zc_ref[row, pl.ds(l * 16, 16)]
                for l in range(nseg)
            ]
            while len(prods) > 1:
                prods = [
                    prods[i] + prods[i + 1] for i in range(0, len(prods), 2)
                ]
            pbuf[pl.ds(e * BLK, BLK)] = prods[0]
        acc = jnp.zeros((16,), jnp.float32)
        for l in range(BLK):
            acc = acc + plsc.load_gather(pbuf, [col_ids[l]])
        out_ref[pl.ds(b * BLK, BLK)] = 1.0 / (1.0 + jnp.exp(-acc))
        return carry

    lax.fori_loop(0, nblk, block, 0)


def _sc_body(z_hbm, eidx_hbm, out_hbm, *scr):
    D = DEPTH
    idx2 = scr[0:D]
    zr = scr[D:2 * D]
    zc = scr[2 * D:3 * D]
    outv = scr[3 * D:4 * D]
    pbuf = scr[4 * D]
    sem_i = scr[4 * D + 1:4 * D + 1 + D]
    sem_g = scr[4 * D + 1 + D:4 * D + 1 + 2 * D]
    sem_o = scr[4 * D + 1 + 2 * D:4 * D + 1 + 3 * D]

    wid = lax.axis_index("c") * NS + lax.axis_index("s")
    base = wid * EDGES_PER_W

    def chunk_base(c):
        eb = jnp.minimum(c * CHUNK, EDGES_PER_W - CHUNK)
        return base + eb

    def fire_idx(c, p):
        ebase = chunk_base(c)
        pltpu.async_copy(
            eidx_hbm.at[:, pl.ds(ebase, CHUNK)], idx2[p], sem_i[p])

    def wait_idx(c, p):
        ebase = chunk_base(c)
        pltpu.make_async_copy(
            eidx_hbm.at[:, pl.ds(ebase, CHUNK)], idx2[p], sem_i[p]).wait()

    def fire_gather(p):
        pltpu.async_copy(z_hbm.at[idx2[p].at[0]], zr[p], sem_g[p])
        pltpu.async_copy(z_hbm.at[idx2[p].at[1]], zc[p], sem_g[p])

    def wait_gather(p):
        pltpu.make_async_copy(z_hbm.at[idx2[p].at[0]], zr[p], sem_g[p]).wait()
        pltpu.make_async_copy(z_hbm.at[idx2[p].at[1]], zc[p], sem_g[p]).wait()

    def fire_out(c, p):
        ebase = chunk_base(c)
        pltpu.async_copy(outv[p], out_hbm.at[pl.ds(ebase, CHUNK)], sem_o[p])

    def wait_out(c, p):
        ebase = chunk_base(c)
        pltpu.make_async_copy(
            outv[p], out_hbm.at[pl.ds(ebase, CHUNK)], sem_o[p]).wait()

    def step(c, p):
        wait_gather(p)

        @pl.when(c + DEPTH < N_CHUNKS)
        def _():
            fire_idx(c + DEPTH, p)

        nxt = (p + DEPTH - 1) % DEPTH

        @pl.when(c + DEPTH - 1 < N_CHUNKS)
        def _():
            wait_idx(c + DEPTH - 1, nxt)
            fire_gather(nxt)

        @pl.when(c >= DEPTH)
        def _():
            wait_out(c - DEPTH, p)

        _dot_chunk(zr[p], zc[p], pbuf, outv[p])
        fire_out(c, p)

    for p in range(DEPTH):
        fire_idx(p, p)
    for p in range(DEPTH - 1):
        wait_idx(p, p)
        fire_gather(p)

    def ring(k, carry):
        c = k * DEPTH
        for p in range(DEPTH):
            step(c + p, p)
        return carry

    lax.fori_loop(0, N_CHUNKS // DEPTH, ring, 0)
    step(N_CHUNKS - 1, 0)

    for t in range(DEPTH):
        c = N_CHUNKS - 1 - t
        wait_out(c, c % DEPTH)


@jax.jit
def _decode(z, eidx):
    mesh = plsc.VectorSubcoreMesh(core_axis_name="c", subcore_axis_name="s")
    f = pl.kernel(
        _sc_body,
        mesh=mesh,
        compiler_params=pltpu.CompilerParams(
            use_tc_tiling_on_sc=False, needs_layout_passes=False
        ),
        out_type=jax.ShapeDtypeStruct((N_EDGES,), jnp.float32),
        scratch_types=(
            [pltpu.VMEM((2, CHUNK), jnp.int32) for _ in range(DEPTH)]
            + [pltpu.VMEM((CHUNK, DIM), jnp.float32) for _ in range(2 * DEPTH)]
            + [pltpu.VMEM((CHUNK,), jnp.float32) for _ in range(DEPTH)]
            + [pltpu.VMEM((BLK * BLK,), jnp.float32)]
            + [pltpu.SemaphoreType.DMA for _ in range(3 * DEPTH)]
        ),
    )
    return f(z, eidx)


def kernel(z, edge_index):
    return _decode(z, edge_index.astype(jnp.int32))


# speedup vs baseline: 1.0091x; 1.0091x over previous
import jax
import jax.numpy as jnp
from jax import lax
from jax.experimental import pallas as pl
from jax.experimental.pallas import tpu as pltpu
from jax.experimental.pallas import tpu_sc as plsc

N_NODES = 10000
DIM = 128
N_EDGES = 320000

NC = 2
NS = 16
NW = NC * NS

EDGES_PER_W = N_EDGES // NW
CHUNK = 128
N_CHUNKS = (EDGES_PER_W + CHUNK - 1) // CHUNK
BLK = 16
DEPTH = 2


def _dot_chunk(zr_ref, zc_ref, pbuf, out_ref):
    nblk = CHUNK // BLK
    nseg = DIM // 16
    col_ids = [lax.iota(jnp.int32, 16) * BLK + l for l in range(BLK)]

    def block(b, carry):
        base_e = b * BLK
        for e in range(BLK):
            row = base_e + e
            prods = [
                zr_ref[row, pl.ds(l * 16, 16)] * zc_ref[row, pl.ds(l * 16, 16)]
                for l in range(nseg)
            ]
            while len(prods) > 1:
                prods = [
                    prods[i] + prods[i + 1] for i in range(0, len(prods), 2)
                ]
            pbuf[pl.ds(e * BLK, BLK)] = prods[0]
        acc = jnp.zeros((16,), jnp.float32)
        for l in range(BLK):
            acc = acc + plsc.load_gather(pbuf, [col_ids[l]])
        out_ref[pl.ds(b * BLK, BLK)] = 1.0 / (1.0 + jnp.exp(-acc))
        return carry

    lax.fori_loop(0, nblk, block, 0)


def _sc_body(z_hbm, eidx_hbm, out_hbm, *scr):
    D = DEPTH
    idx2 = scr[0:D]
    zr = scr[D:2 * D]
    zc = scr[2 * D:3 * D]
    outv = scr[3 * D:4 * D]
    pbuf = scr[4 * D]
    sem_i = scr[4 * D + 1:4 * D + 1 + D]
    sem_g = scr[4 * D + 1 + D:4 * D + 1 + 2 * D]
    sem_o = scr[4 * D + 1 + 2 * D:4 * D + 1 + 3 * D]

    wid = lax.axis_index("c") * NS + lax.axis_index("s")
    base = wid * EDGES_PER_W

    def chunk_base(c):
        eb = jnp.minimum(c * CHUNK, EDGES_PER_W - CHUNK)
        return base + eb

    def fire_idx(c, p):
        ebase = chunk_base(c)
        pltpu.async_copy(
            eidx_hbm.at[:, pl.ds(ebase, CHUNK)], idx2[p], sem_i[p])

    def wait_idx(c, p):
        ebase = chunk_base(c)
        pltpu.make_async_copy(
            eidx_hbm.at[:, pl.ds(ebase, CHUNK)], idx2[p], sem_i[p]).wait()

    def fire_gather(p):
        pltpu.async_copy(z_hbm.at[idx2[p].at[0]], zr[p], sem_g[p])
        pltpu.async_copy(z_hbm.at[idx2[p].at[1]], zc[p], sem_g[p])

    def wait_gather(p):
        pltpu.make_async_copy(z_hbm.at[idx2[p].at[0]], zr[p], sem_g[p]).wait()
        pltpu.make_async_copy(z_hbm.at[idx2[p].at[1]], zc[p], sem_g[p]).wait()

    def fire_out(c, p):
        ebase = chunk_base(c)
        pltpu.async_copy(outv[p], out_hbm.at[pl.ds(ebase, CHUNK)], sem_o[p])

    def wait_out(c, p):
        ebase = chunk_base(c)
        pltpu.make_async_copy(
            outv[p], out_hbm.at[pl.ds(ebase, CHUNK)], sem_o[p]).wait()

    def step(c, p):
        wait_gather(p)

        @pl.when(c + DEPTH < N_CHUNKS)
        def _():
            fire_idx(c + DEPTH, p)

        nxt = (p + DEPTH - 1) % DEPTH

        @pl.when(c + DEPTH - 1 < N_CHUNKS)
        def _():
            wait_idx(c + DEPTH - 1, nxt)
            fire_gather(nxt)

        @pl.when(c >= DEPTH)
        def _():
            wait_out(c - DEPTH, p)

        _dot_chunk(zr[p], zc[p], pbuf, outv[p])
        fire_out(c, p)

    for p in range(DEPTH):
        fire_idx(p, p)
    for p in range(DEPTH - 1):
        wait_idx(p, p)
        fire_gather(p)

    def ring(k, carry):
        c = k * DEPTH
        for p in range(DEPTH):
            step(c + p, p)
        return carry

    lax.fori_loop(0, N_CHUNKS // DEPTH, ring, 0)
    step(N_CHUNKS - 1, 0)

    for t in range(DEPTH):
        c = N_CHUNKS - 1 - t
        wait_out(c, c % DEPTH)


@jax.jit
def _decode(z, eidx):
    mesh = plsc.VectorSubcoreMesh(core_axis_name="c", subcore_axis_name="s")
    f = pl.kernel(
        _sc_body,
        mesh=mesh,
        compiler_params=pltpu.CompilerParams(
            use_tc_tiling_on_sc=False, needs_layout_passes=False
        ),
        out_type=jax.ShapeDtypeStruct((N_EDGES,), jnp.float32),
        scratch_types=(
            [pltpu.VMEM((2, CHUNK), jnp.int32) for _ in range(DEPTH)]
            + [pltpu.VMEM((CHUNK, DIM), jnp.float32) for _ in range(2 * DEPTH)]
            + [pltpu.VMEM((CHUNK,), jnp.float32) for _ in range(DEPTH)]
            + [pltpu.VMEM((BLK * BLK,), jnp.float32)]
            + [pltpu.SemaphoreType.DMA for _ in range(3 * DEPTH)]
        ),
    )
    return f(z, eidx)


def kernel(z, edge_index):
    return _decode(z, edge_index.astype(jnp.int32))
